# Initial kernel scaffold; baseline (speedup 1.0000x reference)
import jax
import jax.numpy as jnp
from jax.experimental import pallas as pl


def kernel(input_embedding, fai_x, fai_x_prime, w_1, b_1, w_2, b_2, Wq, Wk, Wv, Wadd, badd, Wfin, bfin):
    raise NotImplementedError("write your pallas kernel here")


# speedup vs baseline: 8.3488x; 8.3488x over previous
import math

import jax
import jax.numpy as jnp
import numpy as np
from jax.experimental import pallas as pl

N = 2048
D = 768
U = 160
NCLS = 16


def _threefry2x32(k1, k2, x0, x1):
    rots = ((13, 15, 26, 6), (17, 29, 16, 24))
    ks = (np.uint32(k1), np.uint32(k2),
          np.uint32(k1) ^ np.uint32(k2) ^ np.uint32(0x1BD11BDA))
    x0 = x0 + ks[0]
    x1 = x1 + ks[1]
    for i in range(5):
        for r in rots[i % 2]:
            x0 = x0 + x1
            x1 = (x1 << np.uint32(r)) | (x1 >> np.uint32(32 - r))
            x1 = x0 ^ x1
        x0 = x0 + ks[(i + 1) % 3]
        x1 = x1 + ks[(i + 2) % 3] + np.uint32(i + 1)
    return x0, x1


def _index_sample_constant():
    old = np.seterr(over="ignore")
    try:
        sk_hi, sk_lo = _threefry2x32(
            0, 42, np.zeros(2, np.uint32), np.arange(2, dtype=np.uint32))
        k1, k2 = sk_hi[1], sk_lo[1]
        size = N * U
        hb, lb = _threefry2x32(
            k1, k2, np.zeros(size, np.uint32), np.arange(size, dtype=np.uint32))
        bits = hb ^ lb
        return (bits % np.uint32(N)).astype(np.int64).reshape(N, U)
    finally:
        np.seterr(**old)


_idx = _index_sample_constant()
_cnt_np = np.zeros((N, N), np.uint8)
np.add.at(_cnt_np, (np.arange(N)[:, None], _idx), 1)
_CNT = _cnt_np

_QB = 256
_FB = 128


def _proj_body(x_ref, w_ref, o_ref):
    o_ref[...] = jnp.dot(x_ref[...], w_ref[...],
                         preferred_element_type=jnp.float32)


def _vmean_body(v_ref, o_ref):
    o_ref[...] = jnp.mean(v_ref[...], axis=0, keepdims=True)


def _attn_body(q_ref, k_ref, v_ref, cnt_ref, m_ref, ctx_ref):
    q = q_ref[...]
    k = k_ref[...]
    s = jax.lax.dot_general(q, k, (((1,), (1,)), ((), ())),
                            preferred_element_type=jnp.float32)
    cnt = cnt_ref[...].astype(jnp.float32)
    mmax = jnp.max(jnp.where(cnt > 0.0, s, -jnp.inf), axis=1)
    msum = jnp.sum(s * cnt, axis=1)
    m_ref[...] = (mmax - msum * (1.0 / N)).reshape(1, 1, _QB)
    ss = s * (1.0 / math.sqrt(D))
    rm = jnp.max(ss, axis=1, keepdims=True)
    e = jnp.exp(ss - rm)
    p = e / jnp.sum(e, axis=1, keepdims=True)
    ctx_ref[...] = jnp.dot(p, v_ref[...], preferred_element_type=jnp.float32)


def _sel_body(m_ref, sel_ref):
    m = m_ref[...]
    u = jax.lax.bitcast_convert_type(m, jnp.int32)
    key = jnp.where(u < 0, u ^ jnp.int32(0x7FFFFFFF), u)
    row = jax.lax.broadcasted_iota(jnp.int32, (16, 128), 0)
    col = jax.lax.broadcasted_iota(jnp.int32, (16, 128), 1)
    idx = row * 128 + col

    def t_body(b, t):
        tp = t + jnp.left_shift(jnp.int32(1), 31 - b)
        c = jnp.sum((key >= tp).astype(jnp.int32))
        return jnp.where(c >= U, tp, t)

    t = jax.lax.fori_loop(0, 32, t_body, jnp.int32(-2147483647 - 1))
    gt = key > t
    eq = key == t
    need = U - jnp.sum(gt.astype(jnp.int32))

    def j_body(b, j):
        jp = j + jnp.left_shift(jnp.int32(1), 10 - b)
        c = jnp.sum((eq & (idx <= jp)).astype(jnp.int32))
        return jnp.where(c <= need, jp, j)

    j = jax.lax.fori_loop(0, 11, j_body, jnp.int32(-1))
    sel = gt | (eq & (idx <= j))
    sel_ref[...] = sel.astype(jnp.float32)


def _final_body(ctx_ref, add_ref, sel_ref, vm_ref, badd_ref, w_ref, o_ref):
    sel = sel_ref[:, 0:1]
    ctx = ctx_ref[...] * sel + vm_ref[...] * (1.0 - sel)
    ctx = ctx + add_ref[...] + badd_ref[...]
    w = w_ref[...]
    contrib = jnp.sum(w * ctx[None, :, :], axis=(1, 2))

    @pl.when(pl.program_id(0) == 0)
    def _():
        o_ref[...] = jnp.zeros_like(o_ref)

    o_ref[...] += contrib.reshape(1, NCLS)


def kernel(input_embedding, fai_x, fai_x_prime, w_1, b_1, w_2, b_2,
           Wq, Wk, Wv, Wadd, badd, Wfin, bfin):
    x = input_embedding[0]
    wall = jnp.concatenate([Wq, Wk, Wv, Wadd], axis=0).T

    qkva = pl.pallas_call(
        _proj_body,
        grid=(N // _QB, 4),
        in_specs=[
            pl.BlockSpec((_QB, D), lambda i, j: (i, 0)),
            pl.BlockSpec((D, D), lambda i, j: (0, j)),
        ],
        out_specs=pl.BlockSpec((_QB, D), lambda i, j: (i, j)),
        out_shape=jax.ShapeDtypeStruct((N, 4 * D), jnp.float32),
    )(x, wall)
    q = qkva[:, 0 * D:1 * D]
    k = qkva[:, 1 * D:2 * D]
    v = qkva[:, 2 * D:3 * D]
    add = qkva[:, 3 * D:4 * D]

    vmean = pl.pallas_call(
        _vmean_body,
        out_shape=jax.ShapeDtypeStruct((1, D), jnp.float32),
    )(v)

    m3, ctx_attn = pl.pallas_call(
        _attn_body,
        grid=(N // _QB,),
        in_specs=[
            pl.BlockSpec((_QB, D), lambda i: (i, 0)),
            pl.BlockSpec((N, D), lambda i: (0, 0)),
            pl.BlockSpec((N, D), lambda i: (0, 0)),
            pl.BlockSpec((_QB, N), lambda i: (i, 0)),
        ],
        out_specs=[
            pl.BlockSpec((1, 1, _QB), lambda i: (i, 0, 0)),
            pl.BlockSpec((_QB, D), lambda i: (i, 0)),
        ],
        out_shape=[
            jax.ShapeDtypeStruct((N // _QB, 1, _QB), jnp.float32),
            jax.ShapeDtypeStruct((N, D), jnp.float32),
        ],
    )(q, k, v, _CNT)

    sel = pl.pallas_call(
        _sel_body,
        out_shape=jax.ShapeDtypeStruct((16, 128), jnp.float32),
    )(m3.reshape(16, 128))

    selb = jnp.broadcast_to(sel.reshape(N, 1), (N, 128))
    wfin3 = Wfin.reshape(NCLS, N, D)

    out = pl.pallas_call(
        _final_body,
        grid=(N // _FB,),
        in_specs=[
            pl.BlockSpec((_FB, D), lambda i: (i, 0)),
            pl.BlockSpec((_FB, D), lambda i: (i, 0)),
            pl.BlockSpec((_FB, 128), lambda i: (i, 0)),
            pl.BlockSpec((1, D), lambda i: (0, 0)),
            pl.BlockSpec((1, D), lambda i: (0, 0)),
            pl.BlockSpec((NCLS, _FB, D), lambda i: (0, i, 0)),
        ],
        out_specs=pl.BlockSpec((1, NCLS), lambda i: (0, 0)),
        out_shape=jax.ShapeDtypeStruct((1, NCLS), jnp.float32),
    )(ctx_attn, add, selb, vmean, badd.reshape(1, D), wfin3)

    return out + bfin[None, :]


# speedup vs baseline: 12.7494x; 1.5271x over previous
import math

import jax
import jax.numpy as jnp
import numpy as np
from jax.experimental import pallas as pl

N = 2048
D = 768
U = 160
NCLS = 16


def _threefry2x32(k1, k2, x0, x1):
    rots = ((13, 15, 26, 6), (17, 29, 16, 24))
    ks = (np.uint32(k1), np.uint32(k2),
          np.uint32(k1) ^ np.uint32(k2) ^ np.uint32(0x1BD11BDA))
    x0 = x0 + ks[0]
    x1 = x1 + ks[1]
    for i in range(5):
        for r in rots[i % 2]:
            x0 = x0 + x1
            x1 = (x1 << np.uint32(r)) | (x1 >> np.uint32(32 - r))
            x1 = x0 ^ x1
        x0 = x0 + ks[(i + 1) % 3]
        x1 = x1 + ks[(i + 2) % 3] + np.uint32(i + 1)
    return x0, x1


def _index_sample_constant():
    old = np.seterr(over="ignore")
    try:
        sk_hi, sk_lo = _threefry2x32(
            0, 42, np.zeros(2, np.uint32), np.arange(2, dtype=np.uint32))
        k1, k2 = sk_hi[1], sk_lo[1]
        size = N * U
        hb, lb = _threefry2x32(
            k1, k2, np.zeros(size, np.uint32), np.arange(size, dtype=np.uint32))
        bits = hb ^ lb
        return (bits % np.uint32(N)).astype(np.int64).reshape(N, U)
    finally:
        np.seterr(**old)


_idx = _index_sample_constant()
_cnt_np = np.zeros((N, N), np.uint8)
np.add.at(_cnt_np, (np.arange(N)[:, None], _idx), 1)
_CNT = _cnt_np

_QB = 256
_FB = 128


def _proj_body(x_ref, wq_ref, wk_ref, wv_ref, wa_ref,
               q_ref, k_ref, v_ref, a_ref):
    x = x_ref[...]
    dn = (((1,), (1,)), ((), ()))
    q_ref[...] = jax.lax.dot_general(x, wq_ref[...], dn,
                                     preferred_element_type=jnp.float32)
    k_ref[...] = jax.lax.dot_general(x, wk_ref[...], dn,
                                     preferred_element_type=jnp.float32)
    v_ref[...] = jax.lax.dot_general(x, wv_ref[...], dn,
                                     preferred_element_type=jnp.float32)
    a_ref[...] = jax.lax.dot_general(x, wa_ref[...], dn,
                                     preferred_element_type=jnp.float32)


def _vmean_body(v_ref, o_ref):
    o_ref[...] = jnp.mean(v_ref[...], axis=0, keepdims=True)


def _attn_body(q_ref, k_ref, v_ref, cnt_ref, m_ref, ctx_ref):
    q = q_ref[...]
    k = k_ref[...]
    s = jax.lax.dot_general(q, k, (((1,), (1,)), ((), ())),
                            preferred_element_type=jnp.float32)
    cnt = cnt_ref[...].astype(jnp.float32)
    mmax = jnp.max(jnp.where(cnt > 0.0, s, -jnp.inf), axis=1)
    msum = jnp.sum(s * cnt, axis=1)
    m_ref[...] = (mmax - msum * (1.0 / N)).reshape(1, 1, _QB)
    ss = s * (1.0 / math.sqrt(D))
    rm = jnp.max(ss, axis=1, keepdims=True)
    e = jnp.exp(ss - rm)
    p = e / jnp.sum(e, axis=1, keepdims=True)
    ctx_ref[...] = jnp.dot(p, v_ref[...], preferred_element_type=jnp.float32)


def _sel_body(m_ref, sel_ref):
    m = m_ref[...]
    u = jax.lax.bitcast_convert_type(m, jnp.int32)
    key = jnp.where(u < 0, u ^ jnp.int32(0x7FFFFFFF), u)
    row = jax.lax.broadcasted_iota(jnp.int32, (16, 128), 0)
    col = jax.lax.broadcasted_iota(jnp.int32, (16, 128), 1)
    idx = row * 128 + col

    def t_body(b, t):
        tp = t + jnp.left_shift(jnp.int32(1), 31 - b)
        c = jnp.sum((key >= tp).astype(jnp.int32))
        return jnp.where(c >= U, tp, t)

    t = jax.lax.fori_loop(0, 32, t_body, jnp.int32(-2147483647 - 1))
    gt = key > t
    eq = key == t
    need = U - jnp.sum(gt.astype(jnp.int32))

    def j_body(b, j):
        jp = j + jnp.left_shift(jnp.int32(1), 10 - b)
        c = jnp.sum((eq & (idx <= jp)).astype(jnp.int32))
        return jnp.where(c <= need, jp, j)

    j = jax.lax.fori_loop(0, 11, j_body, jnp.int32(-1))
    sel = gt | (eq & (idx <= j))
    sel_ref[...] = sel.astype(jnp.float32)


def _final_body(ctx_ref, add_ref, sel_ref, vm_ref, badd_ref, w_ref, o_ref):
    sel = sel_ref[:, 0:1]
    ctx = ctx_ref[...] * sel + vm_ref[...] * (1.0 - sel)
    ctx = ctx + add_ref[...] + badd_ref[...]
    w = w_ref[...]
    contrib = jnp.sum(w * ctx[None, :, :], axis=(1, 2))

    @pl.when(pl.program_id(0) == 0)
    def _():
        o_ref[...] = jnp.zeros_like(o_ref)

    o_ref[...] += contrib.reshape(1, NCLS)


def kernel(input_embedding, fai_x, fai_x_prime, w_1, b_1, w_2, b_2,
           Wq, Wk, Wv, Wadd, badd, Wfin, bfin):
    x = input_embedding.reshape(N, D)

    q, k, v, add = pl.pallas_call(
        _proj_body,
        grid=(N // _QB,),
        in_specs=[
            pl.BlockSpec((_QB, D), lambda i: (i, 0)),
            pl.BlockSpec((D, D), lambda i: (0, 0)),
            pl.BlockSpec((D, D), lambda i: (0, 0)),
            pl.BlockSpec((D, D), lambda i: (0, 0)),
            pl.BlockSpec((D, D), lambda i: (0, 0)),
        ],
        out_specs=[pl.BlockSpec((_QB, D), lambda i: (i, 0))] * 4,
        out_shape=[jax.ShapeDtypeStruct((N, D), jnp.float32)] * 4,
    )(x, Wq, Wk, Wv, Wadd)

    vmean = pl.pallas_call(
        _vmean_body,
        out_shape=jax.ShapeDtypeStruct((1, D), jnp.float32),
    )(v)

    m3, ctx_attn = pl.pallas_call(
        _attn_body,
        grid=(N // _QB,),
        in_specs=[
            pl.BlockSpec((_QB, D), lambda i: (i, 0)),
            pl.BlockSpec((N, D), lambda i: (0, 0)),
            pl.BlockSpec((N, D), lambda i: (0, 0)),
            pl.BlockSpec((_QB, N), lambda i: (i, 0)),
        ],
        out_specs=[
            pl.BlockSpec((1, 1, _QB), lambda i: (i, 0, 0)),
            pl.BlockSpec((_QB, D), lambda i: (i, 0)),
        ],
        out_shape=[
            jax.ShapeDtypeStruct((N // _QB, 1, _QB), jnp.float32),
            jax.ShapeDtypeStruct((N, D), jnp.float32),
        ],
    )(q, k, v, _CNT)

    sel = pl.pallas_call(
        _sel_body,
        out_shape=jax.ShapeDtypeStruct((16, 128), jnp.float32),
    )(m3.reshape(16, 128))

    selb = jnp.broadcast_to(sel.reshape(N, 1), (N, 128))
    wfin3 = Wfin.reshape(NCLS, N, D)

    out = pl.pallas_call(
        _final_body,
        grid=(N // _FB,),
        in_specs=[
            pl.BlockSpec((_FB, D), lambda i: (i, 0)),
            pl.BlockSpec((_FB, D), lambda i: (i, 0)),
            pl.BlockSpec((_FB, 128), lambda i: (i, 0)),
            pl.BlockSpec((1, D), lambda i: (0, 0)),
            pl.BlockSpec((1, D), lambda i: (0, 0)),
            pl.BlockSpec((NCLS, _FB, D), lambda i: (0, i, 0)),
        ],
        out_specs=pl.BlockSpec((1, NCLS), lambda i: (0, 0)),
        out_shape=jax.ShapeDtypeStruct((1, NCLS), jnp.float32),
    )(ctx_attn, add, selb, vmean, badd.reshape(1, D), wfin3)

    return out + bfin[None, :]


# speedup vs baseline: 15.7267x; 1.2335x over previous
import math

import jax
import jax.numpy as jnp
import numpy as np
from jax.experimental import pallas as pl

N = 2048
D = 768
U = 160
NCLS = 16


def _threefry2x32(k1, k2, x0, x1):
    rots = ((13, 15, 26, 6), (17, 29, 16, 24))
    ks = (np.uint32(k1), np.uint32(k2),
          np.uint32(k1) ^ np.uint32(k2) ^ np.uint32(0x1BD11BDA))
    x0 = x0 + ks[0]
    x1 = x1 + ks[1]
    for i in range(5):
        for r in rots[i % 2]:
            x0 = x0 + x1
            x1 = (x1 << np.uint32(r)) | (x1 >> np.uint32(32 - r))
            x1 = x0 ^ x1
        x0 = x0 + ks[(i + 1) % 3]
        x1 = x1 + ks[(i + 2) % 3] + np.uint32(i + 1)
    return x0, x1


def _index_sample_constant():
    old = np.seterr(over="ignore")
    try:
        sk_hi, sk_lo = _threefry2x32(
            0, 42, np.zeros(2, np.uint32), np.arange(2, dtype=np.uint32))
        k1, k2 = sk_hi[1], sk_lo[1]
        size = N * U
        hb, lb = _threefry2x32(
            k1, k2, np.zeros(size, np.uint32), np.arange(size, dtype=np.uint32))
        bits = hb ^ lb
        return (bits % np.uint32(N)).astype(np.int64).reshape(N, U)
    finally:
        np.seterr(**old)


_idx = _index_sample_constant()
_cnt_np = np.zeros((N, N), np.uint8)
np.add.at(_cnt_np, (np.arange(N)[:, None], _idx), 1)
_CNT = _cnt_np

_QB = 256
_FB = 128


def _proj_body(x_ref, wq_ref, wk_ref, wv_ref, wa_ref,
               q_ref, k_ref, v_ref, a_ref):
    x = x_ref[...]
    dn = (((1,), (1,)), ((), ()))
    q_ref[...] = jax.lax.dot_general(x, wq_ref[...], dn,
                                     preferred_element_type=jnp.float32)
    k_ref[...] = jax.lax.dot_general(x, wk_ref[...], dn,
                                     preferred_element_type=jnp.float32)
    v_ref[...] = jax.lax.dot_general(x, wv_ref[...], dn,
                                     preferred_element_type=jnp.float32)
    a_ref[...] = jax.lax.dot_general(x, wa_ref[...], dn,
                                     preferred_element_type=jnp.float32)


def _vmean_body(v_ref, o_ref):
    o_ref[...] = jnp.mean(v_ref[...], axis=0, keepdims=True)


def _attn_body(q_ref, k_ref, v_ref, cnt_ref, m_ref, ctx_ref):
    q = q_ref[...]
    k = k_ref[...]
    s = jax.lax.dot_general(q, k, (((1,), (1,)), ((), ())),
                            preferred_element_type=jnp.float32)
    cnt = cnt_ref[...].astype(jnp.float32)
    mmax = jnp.max(jnp.where(cnt > 0.0, s, -jnp.inf), axis=1)
    msum = jnp.sum(s * cnt, axis=1)
    m_ref[...] = (mmax - msum * (1.0 / N)).reshape(1, 1, _QB)
    ss = s * (1.0 / math.sqrt(D))
    rm = jnp.max(ss, axis=1, keepdims=True)
    e = jnp.exp(ss - rm)
    p = e / jnp.sum(e, axis=1, keepdims=True)
    ctx_ref[...] = jnp.dot(p, v_ref[...], preferred_element_type=jnp.float32)


def _sel_body(m_ref, sel_ref):
    m = m_ref[...]
    u = jax.lax.bitcast_convert_type(m, jnp.int32)
    key = jnp.where(u < 0, u ^ jnp.int32(0x7FFFFFFF), u)
    row = jax.lax.broadcasted_iota(jnp.int32, (16, 128), 0)
    col = jax.lax.broadcasted_iota(jnp.int32, (16, 128), 1)
    idx = row * 128 + col

    def t_body(b, t):
        tp = t + jnp.left_shift(jnp.int32(1), 31 - b)
        c = jnp.sum((key >= tp).astype(jnp.int32))
        return jnp.where(c >= U, tp, t)

    t = jax.lax.fori_loop(0, 32, t_body, jnp.int32(-2147483647 - 1))
    gt = key > t
    eq = key == t
    need = U - jnp.sum(gt.astype(jnp.int32))

    def j_body(b, j):
        jp = j + jnp.left_shift(jnp.int32(1), 10 - b)
        c = jnp.sum((eq & (idx <= jp)).astype(jnp.int32))
        return jnp.where(c <= need, jp, j)

    j = jax.lax.fori_loop(0, 11, j_body, jnp.int32(-1))
    sel = gt | (eq & (idx <= j))
    sel_ref[...] = sel.astype(jnp.float32)


def _blend_body(ctx_ref, add_ref, sel_ref, vm_ref, badd_ref, o_ref):
    sel = sel_ref[:, 0:1]
    ctx = ctx_ref[...] * sel + vm_ref[...] * (1.0 - sel)
    o_ref[...] = ctx + add_ref[...] + badd_ref[...]


def _final_body(ctxf_ref, w_ref, o_ref):
    i = pl.program_id(0)
    crow = ctxf_ref[pl.ds(i, 1), :]
    w = w_ref[...]
    contrib = jnp.sum(w * crow, axis=1)

    @pl.when(i == 0)
    def _():
        o_ref[...] = jnp.zeros_like(o_ref)

    o_ref[...] += contrib.reshape(1, NCLS)


def kernel(input_embedding, fai_x, fai_x_prime, w_1, b_1, w_2, b_2,
           Wq, Wk, Wv, Wadd, badd, Wfin, bfin):
    x = input_embedding.reshape(N, D)

    q, k, v, add = pl.pallas_call(
        _proj_body,
        grid=(N // _QB,),
        in_specs=[
            pl.BlockSpec((_QB, D), lambda i: (i, 0)),
            pl.BlockSpec((D, D), lambda i: (0, 0)),
            pl.BlockSpec((D, D), lambda i: (0, 0)),
            pl.BlockSpec((D, D), lambda i: (0, 0)),
            pl.BlockSpec((D, D), lambda i: (0, 0)),
        ],
        out_specs=[pl.BlockSpec((_QB, D), lambda i: (i, 0))] * 4,
        out_shape=[jax.ShapeDtypeStruct((N, D), jnp.float32)] * 4,
    )(x, Wq, Wk, Wv, Wadd)

    vmean = pl.pallas_call(
        _vmean_body,
        out_shape=jax.ShapeDtypeStruct((1, D), jnp.float32),
    )(v)

    m3, ctx_attn = pl.pallas_call(
        _attn_body,
        grid=(N // _QB,),
        in_specs=[
            pl.BlockSpec((_QB, D), lambda i: (i, 0)),
            pl.BlockSpec((N, D), lambda i: (0, 0)),
            pl.BlockSpec((N, D), lambda i: (0, 0)),
            pl.BlockSpec((_QB, N), lambda i: (i, 0)),
        ],
        out_specs=[
            pl.BlockSpec((1, 1, _QB), lambda i: (i, 0, 0)),
            pl.BlockSpec((_QB, D), lambda i: (i, 0)),
        ],
        out_shape=[
            jax.ShapeDtypeStruct((N // _QB, 1, _QB), jnp.float32),
            jax.ShapeDtypeStruct((N, D), jnp.float32),
        ],
    )(q, k, v, _CNT)

    sel = pl.pallas_call(
        _sel_body,
        out_shape=jax.ShapeDtypeStruct((16, 128), jnp.float32),
    )(m3.reshape(16, 128))

    selb = jnp.broadcast_to(sel.reshape(N, 1), (N, 128))

    ctx_final = pl.pallas_call(
        _blend_body,
        grid=(N // _FB,),
        in_specs=[
            pl.BlockSpec((_FB, D), lambda i: (i, 0)),
            pl.BlockSpec((_FB, D), lambda i: (i, 0)),
            pl.BlockSpec((_FB, 128), lambda i: (i, 0)),
            pl.BlockSpec((1, D), lambda i: (0, 0)),
            pl.BlockSpec((1, D), lambda i: (0, 0)),
        ],
        out_specs=pl.BlockSpec((_FB, D), lambda i: (i, 0)),
        out_shape=jax.ShapeDtypeStruct((N, D), jnp.float32),
    )(ctx_attn, add, selb, vmean, badd.reshape(1, D))

    chunk = N * D // 16
    ctxf = ctx_final.reshape(16, chunk)

    out = pl.pallas_call(
        _final_body,
        grid=(16,),
        in_specs=[
            pl.BlockSpec((16, chunk), lambda i: (0, 0)),
            pl.BlockSpec((NCLS, chunk), lambda i: (0, i)),
        ],
        out_specs=pl.BlockSpec((1, NCLS), lambda i: (0, 0)),
        out_shape=jax.ShapeDtypeStruct((1, NCLS), jnp.float32),
    )(ctxf, Wfin)

    return out + bfin[None, :]
